# Initial kernel scaffold; baseline (speedup 1.0000x reference)
import jax
import jax.numpy as jnp
from jax.experimental import pallas as pl


def kernel(ts, params):
    raise NotImplementedError("write your pallas kernel here")


# speedup vs baseline: 9.2781x; 9.2781x over previous
import functools

import jax
import jax.numpy as jnp
from jax import lax
from jax.experimental import pallas as pl
from jax.experimental.pallas import tpu as pltpu
from jax.experimental.pallas import tpu_sc as plsc

B, M, T = 4, 38, 2048
C1, C2 = 64, 128
K = 9
EPS_BN = 1e-5
N = T
RT = 256
NT = N // RT
BN_ = B * N
NE = BN_ * K
EC = 16
NCH = BN_ // EC

SC_NC, SC_NS, SC_L = 2, 16, 16
SC_NW = SC_NC * SC_NS

_INTERPRET = False


def _gelu(x):
    return x * 0.5 * (1.0 + lax.erf(x * (2.0 ** -0.5)))


def _bfdot(a, b):
    return jnp.dot(a.astype(jnp.bfloat16), b.astype(jnp.bfloat16),
                   preferred_element_type=jnp.float32)


def _conv_body(x_ref, w1_ref, b1_ref, w2_ref, b2_ref, out_ref, h_ref):
    x = x_ref[0].astype(jnp.bfloat16)
    h = jnp.zeros((T, C1), jnp.float32)
    for j in range(3):
        h = h + jnp.dot(x[6 + j:6 + j + T, :], w1_ref[j].astype(jnp.bfloat16),
                        preferred_element_type=jnp.float32)
    h = jnp.maximum(h + b1_ref[0:1, :], 0.0)
    h_ref[0:8, :] = jnp.zeros((8, 128), jnp.float32)
    h_ref[8:, :] = jnp.concatenate(
        [h, jnp.zeros((T, 64), jnp.float32)], axis=1)
    o = jnp.zeros((T, C2), jnp.float32)
    for j in range(3):
        o = o + jnp.dot(h_ref[4 + 2 * j:4 + 2 * j + T, :].astype(jnp.bfloat16),
                        w2_ref[j].astype(jnp.bfloat16),
                        preferred_element_type=jnp.float32)
    out_ref[0] = jnp.maximum(o + b2_ref[0:1, :], 0.0)


def _conv_call(xpad, w1, b18, w2, b28):
    return pl.pallas_call(
        _conv_body,
        grid=(B,),
        in_specs=[
            pl.BlockSpec((1, T + 8, 128), lambda b: (b, 0, 0)),
            pl.BlockSpec((3, 128, C1), lambda b: (0, 0, 0)),
            pl.BlockSpec((8, C1), lambda b: (0, 0)),
            pl.BlockSpec((3, 128, C2), lambda b: (0, 0, 0)),
            pl.BlockSpec((8, C2), lambda b: (0, 0)),
        ],
        out_specs=pl.BlockSpec((1, T, C2), lambda b: (b, 0, 0)),
        out_shape=jax.ShapeDtypeStruct((B, T, C2), jnp.float32),
        scratch_shapes=[pltpu.VMEM((T + 8, 128), jnp.float32)],
        interpret=_INTERPRET,
    )(xpad, w1, b18, w2, b28)


def _prep_body(f_ref, xn_ref):
    f = f_ref[0]
    nrm = jnp.sqrt(jnp.sum(f * f, axis=1, keepdims=True))
    xn_ref[0] = f / jnp.maximum(nrm, 1e-12)


def _prep_call(feat):
    return pl.pallas_call(
        _prep_body,
        grid=(B,),
        in_specs=[pl.BlockSpec((1, N, C2), lambda b: (b, 0, 0))],
        out_specs=pl.BlockSpec((1, N, C2), lambda b: (b, 0, 0)),
        out_shape=jax.ShapeDtypeStruct((B, N, C2), jnp.float32),
        interpret=_INTERPRET,
    )(feat)


def _topk_body(xt_ref, xf_ref, idx_ref):
    b = pl.program_id(0)
    sim = lax.dot_general(xt_ref[0, 0].astype(jnp.bfloat16),
                          xf_ref[0].astype(jnp.bfloat16),
                          (((1,), (1,)), ((), ())),
                          preferred_element_type=jnp.float32)
    colio = lax.broadcasted_iota(jnp.int32, (RT, N), 1)
    lane16 = lax.broadcasted_iota(jnp.int32, (RT, 16), 1)
    idxacc = jnp.zeros((RT, 16), jnp.int32)
    boff = b * N
    for e in range(K):
        m = jnp.max(sim, axis=1, keepdims=True)
        am = jnp.min(jnp.where(sim >= m, colio, N), axis=1, keepdims=True)
        sim = jnp.where(colio == am, -jnp.inf, sim)
        idxacc = jnp.where(lane16 == e, am + boff, idxacc)
    idx_ref[0, 0] = idxacc


def _topk_call(xn):
    return pl.pallas_call(
        _topk_body,
        grid=(B, NT),
        in_specs=[
            pl.BlockSpec((1, 1, RT, C2), lambda b, t: (b, t, 0, 0)),
            pl.BlockSpec((1, N, C2), lambda b, t: (b, 0, 0)),
        ],
        out_specs=pl.BlockSpec((1, 1, RT, 16), lambda b, t: (b, t, 0, 0)),
        out_shape=jax.ShapeDtypeStruct((B, NT, RT, 16), jnp.int32),
        interpret=_INTERPRET,
    )(xn.reshape(B, NT, RT, C2), xn)


def _gather_diff_sc(feat_flat, idx_flat):
    GN = 8
    GI = GN * K
    PW = BN_ // SC_NW
    NG = PW // GN
    mesh = plsc.VectorSubcoreMesh(core_axis_name="c", subcore_axis_name="s")

    @functools.partial(
        pl.kernel,
        out_type=jax.ShapeDtypeStruct((NE, C2), jnp.float32),
        mesh=mesh,
        scratch_types=[
            pltpu.VMEM((GI,), jnp.int32),
            pltpu.VMEM((GI, C2), jnp.float32),
            pltpu.VMEM((GN, C2), jnp.float32),
            pltpu.SemaphoreType.DMA,
        ],
    )
    def k(feat_hbm, idx_hbm, d_hbm, idx_v, rows_v, xi_v, sem):
        wid = lax.axis_index("s") * SC_NC + lax.axis_index("c")

        def body(g, carry):
            base = wid * PW + g * GN
            pltpu.sync_copy(idx_hbm.at[pl.ds(base * K, GI)], idx_v)
            pltpu.sync_copy(feat_hbm.at[pl.ds(base, GN)], xi_v)
            pltpu.async_copy(feat_hbm.at[idx_v], rows_v, sem).wait()
            for i in range(GN):
                for c in range(C2 // SC_L):
                    sl = pl.ds(c * SC_L, SC_L)
                    xi = xi_v[i, sl]
                    for r in range(K):
                        rows_v[i * K + r, sl] = rows_v[i * K + r, sl] - xi
            pltpu.sync_copy(rows_v, d_hbm.at[pl.ds(base * K, GI)])
            return carry

        lax.fori_loop(0, NG, body, 0)

    return k(feat_flat, idx_flat)


def _edge_ffn_body(f_ref, d_ref, w1e_ref, w2e_ref, eb_ref, w1_ref, sb1_ref,
                   w2_ref, sb2_ref, out_ref):
    f = f_ref[...]
    a1 = _bfdot(f, w1e_ref[...])
    y2 = _bfdot(d_ref[...], w2e_ref[...])
    y = y2.reshape(NCH, K, C2) + a1[:, None, :]
    ebv = eb_ref[...]
    z = y * ebv[0].reshape(1, 1, C2) + ebv[1].reshape(1, 1, C2)
    h = jnp.max(_gelu(z), axis=1)
    enc = f + h
    h1 = _gelu(_bfdot(enc, w1_ref[...]) * sb1_ref[0:1, :] + sb1_ref[1:2, :])
    out_ref[...] = (_bfdot(h1, w2_ref[...]) * sb2_ref[0:1, :]
                    + sb2_ref[1:2, :])


def _edge_ffn_call(feat2, d, w1et, w2et, eb, w1t, sb1, w2t, sb2):
    return pl.pallas_call(
        _edge_ffn_body,
        grid=(EC,),
        in_specs=[
            pl.BlockSpec((NCH, C2), lambda i: (i, 0)),
            pl.BlockSpec((NCH * K, C2), lambda i: (i, 0)),
            pl.BlockSpec((C2, C2), lambda i: (0, 0)),
            pl.BlockSpec((C2, C2), lambda i: (0, 0)),
            pl.BlockSpec((8, C2), lambda i: (0, 0)),
            pl.BlockSpec((C2, C2), lambda i: (0, 0)),
            pl.BlockSpec((8, C2), lambda i: (0, 0)),
            pl.BlockSpec((C2, C2), lambda i: (0, 0)),
            pl.BlockSpec((8, C2), lambda i: (0, 0)),
        ],
        out_specs=pl.BlockSpec((NCH, C2), lambda i: (i, 0)),
        out_shape=jax.ShapeDtypeStruct((BN_, C2), jnp.float32),
        interpret=_INTERPRET,
    )(feat2, d, w1et, w2et, eb, w1t, sb1, w2t, sb2)


def _rows8(*vecs):
    out = jnp.zeros((8, vecs[0].shape[0]), jnp.float32)
    for i, v in enumerate(vecs):
        out = out.at[i, :].set(v)
    return out


def kernel(ts, params):
    p = params
    f32 = jnp.float32
    inv = 1.0 / jnp.sqrt(1.0 + EPS_BN)

    w1 = jnp.zeros((3, 128, C1), f32)
    for j in range(3):
        w1 = w1.at[j, :M, :].set(p["conv1_w"][:, :, j].T)
    w2 = jnp.zeros((3, 128, C2), f32)
    for j in range(3):
        w2 = w2.at[j, :C1, :].set(p["conv2_w"][:, :, j].T)
    b18 = jnp.broadcast_to(p["conv1_b"][None, :], (8, C1))
    b28 = jnp.broadcast_to(p["conv2_b"][None, :], (8, C2))

    xT = jnp.transpose(ts, (0, 2, 1))
    xpad = jnp.zeros((B, T + 8, 128), f32).at[:, 8:, :M].set(xT)

    feat = _conv_call(xpad, w1, b18, w2, b28)

    for i in range(2):
        s = p[f"g{i}_gamma"] * inv
        w1et = p[f"g{i}_We"][:, :C2].T
        w2et = p[f"g{i}_We"][:, C2:].T
        eb = _rows8(s, s * p[f"g{i}_be"] + p[f"g{i}_beta"])
        s1 = p[f"f{i}_g1"] * inv
        s2 = p[f"f{i}_g2"] * inv
        sb1 = _rows8(s1, s1 * p[f"f{i}_b1"] + p[f"f{i}_bt1"])
        sb2 = _rows8(s2, s2 * p[f"f{i}_b2"] + p[f"f{i}_bt2"])

        xn = _prep_call(feat)
        idx4 = _topk_call(xn)
        idx_flat = idx4[..., :K].reshape(NE)
        feat2 = feat.reshape(BN_, C2)
        d = _gather_diff_sc(feat2, idx_flat)
        feat = _edge_ffn_call(feat2, d, w1et, w2et, eb, p[f"f{i}_W1"].T,
                              sb1, p[f"f{i}_W2"].T, sb2).reshape(B, N, C2)

    return feat


# speedup vs baseline: 13.5290x; 1.4582x over previous
import functools

import jax
import jax.numpy as jnp
from jax import lax
from jax.experimental import pallas as pl
from jax.experimental.pallas import tpu as pltpu
from jax.experimental.pallas import tpu_sc as plsc

B, M, T = 4, 38, 2048
C1, C2 = 64, 128
K = 9
EPS_BN = 1e-5
N = T
RT = 256
NT = N // RT
BN_ = B * N
NE = BN_ * K
EC = 16
NCH = BN_ // EC

SC_NC, SC_NS, SC_L = 2, 16, 16
SC_NW = SC_NC * SC_NS

_INTERPRET = False


def _gelu(x):
    return x * 0.5 * (1.0 + lax.erf(x * (2.0 ** -0.5)))


def _bfdot(a, b):
    return jnp.dot(a.astype(jnp.bfloat16), b.astype(jnp.bfloat16),
                   preferred_element_type=jnp.float32)


def _conv_body(x_ref, w1_ref, b1_ref, w2_ref, b2_ref, out_ref, h_ref):
    x = x_ref[0].astype(jnp.bfloat16)
    h = jnp.zeros((T, C1), jnp.float32)
    for j in range(3):
        h = h + jnp.dot(x[6 + j:6 + j + T, :], w1_ref[j].astype(jnp.bfloat16),
                        preferred_element_type=jnp.float32)
    h = jnp.maximum(h + b1_ref[0:1, :], 0.0)
    h_ref[0:8, :] = jnp.zeros((8, 128), jnp.float32)
    h_ref[8:, :] = jnp.concatenate(
        [h, jnp.zeros((T, 64), jnp.float32)], axis=1)
    o = jnp.zeros((T, C2), jnp.float32)
    for j in range(3):
        o = o + jnp.dot(h_ref[4 + 2 * j:4 + 2 * j + T, :].astype(jnp.bfloat16),
                        w2_ref[j].astype(jnp.bfloat16),
                        preferred_element_type=jnp.float32)
    out_ref[0] = jnp.maximum(o + b2_ref[0:1, :], 0.0)


def _conv_call(xpad, w1, b18, w2, b28):
    return pl.pallas_call(
        _conv_body,
        grid=(B,),
        in_specs=[
            pl.BlockSpec((1, T + 8, 128), lambda b: (b, 0, 0)),
            pl.BlockSpec((3, 128, C1), lambda b: (0, 0, 0)),
            pl.BlockSpec((8, C1), lambda b: (0, 0)),
            pl.BlockSpec((3, 128, C2), lambda b: (0, 0, 0)),
            pl.BlockSpec((8, C2), lambda b: (0, 0)),
        ],
        out_specs=pl.BlockSpec((1, T, C2), lambda b: (b, 0, 0)),
        out_shape=jax.ShapeDtypeStruct((B, T, C2), jnp.float32),
        scratch_shapes=[pltpu.VMEM((T + 8, 128), jnp.float32)],
        interpret=_INTERPRET,
    )(xpad, w1, b18, w2, b28)


def _prep_body(f_ref, xn_ref):
    f = f_ref[0]
    nrm = jnp.sqrt(jnp.sum(f * f, axis=1, keepdims=True))
    xn_ref[0] = f / jnp.maximum(nrm, 1e-12)


def _prep_call(feat):
    return pl.pallas_call(
        _prep_body,
        grid=(B,),
        in_specs=[pl.BlockSpec((1, N, C2), lambda b: (b, 0, 0))],
        out_specs=pl.BlockSpec((1, N, C2), lambda b: (b, 0, 0)),
        out_shape=jax.ShapeDtypeStruct((B, N, C2), jnp.float32),
        interpret=_INTERPRET,
    )(feat)


def _topk_body(xt_ref, xf_ref, idx_ref):
    b = pl.program_id(0)
    sim = lax.dot_general(xt_ref[0, 0].astype(jnp.bfloat16),
                          xf_ref[0].astype(jnp.bfloat16),
                          (((1,), (1,)), ((), ())),
                          preferred_element_type=jnp.float32)
    colio = lax.broadcasted_iota(jnp.int32, (RT, N), 1)
    lane16 = lax.broadcasted_iota(jnp.int32, (RT, 16), 1)
    key = (lax.shift_left(((sim + 1.03125) * 262144.0).astype(jnp.int32), 11)
           | (2047 - colio))
    idxacc = jnp.zeros((RT, 16), jnp.int32)
    boff = b * N
    for e in range(K):
        m = jnp.max(key, axis=1, keepdims=True)
        am = 2047 - (m & 2047)
        key = jnp.where(key == m, jnp.int32(-2147483648), key)
        idxacc = jnp.where(lane16 == e, am + boff, idxacc)
    idx_ref[0, 0] = idxacc


def _topk_call(xn):
    return pl.pallas_call(
        _topk_body,
        grid=(B, NT),
        in_specs=[
            pl.BlockSpec((1, 1, RT, C2), lambda b, t: (b, t, 0, 0)),
            pl.BlockSpec((1, N, C2), lambda b, t: (b, 0, 0)),
        ],
        out_specs=pl.BlockSpec((1, 1, RT, 16), lambda b, t: (b, t, 0, 0)),
        out_shape=jax.ShapeDtypeStruct((B, NT, RT, 16), jnp.int32),
        interpret=_INTERPRET,
    )(xn.reshape(B, NT, RT, C2), xn)


def _gather_diff_sc(feat_flat, idx_flat):
    GN = 8
    GI = GN * K
    PW = BN_ // SC_NW
    NG = PW // GN
    mesh = plsc.VectorSubcoreMesh(core_axis_name="c", subcore_axis_name="s")

    @functools.partial(
        pl.kernel,
        out_type=jax.ShapeDtypeStruct((NE, C2), jnp.float32),
        mesh=mesh,
        scratch_types=[
            pltpu.VMEM((GI,), jnp.int32),
            pltpu.VMEM((GI,), jnp.int32),
            pltpu.VMEM((GI, C2), jnp.float32),
            pltpu.VMEM((GI, C2), jnp.float32),
            pltpu.VMEM((GN, C2), jnp.float32),
            pltpu.VMEM((GN, C2), jnp.float32),
            pltpu.SemaphoreType.DMA,
            pltpu.SemaphoreType.DMA,
        ],
    )
    def k(feat_hbm, idx_hbm, d_hbm, idx_v0, idx_v1, rows_v0, rows_v1,
          xi_v0, xi_v1, sem0, sem1):
        wid = lax.axis_index("s") * SC_NC + lax.axis_index("c")
        idx_v = (idx_v0, idx_v1)
        rows_v = (rows_v0, rows_v1)
        xi_v = (xi_v0, xi_v1)
        sem = (sem0, sem1)

        def body(gp, carry):
            copies = []
            for s in range(2):
                base = (wid * PW + (gp * 2 + s) * GN)
                pltpu.sync_copy(idx_hbm.at[pl.ds(base * K, GI)], idx_v[s])
                pltpu.sync_copy(feat_hbm.at[pl.ds(base, GN)], xi_v[s])
                copies.append(pltpu.async_copy(feat_hbm.at[idx_v[s]],
                                               rows_v[s], sem[s]))
            for s in range(2):
                base = (wid * PW + (gp * 2 + s) * GN)
                copies[s].wait()
                for i in range(GN):
                    for c in range(C2 // SC_L):
                        sl = pl.ds(c * SC_L, SC_L)
                        xi = xi_v[s][i, sl]
                        for r in range(K):
                            rows_v[s][i * K + r, sl] = (
                                rows_v[s][i * K + r, sl] - xi)
                pltpu.sync_copy(rows_v[s], d_hbm.at[pl.ds(base * K, GI)])
            return carry

        lax.fori_loop(0, NG // 2, body, 0)

    return k(feat_flat, idx_flat)


def _edge_ffn_body(f_ref, d_ref, w1e_ref, w2e_ref, eb_ref, w1_ref, sb1_ref,
                   w2_ref, sb2_ref, out_ref):
    f = f_ref[...]
    a1 = _bfdot(f, w1e_ref[...])
    y2 = _bfdot(d_ref[...], w2e_ref[...])
    y2r = y2.reshape(NCH, K, C2)
    ymx = jnp.max(y2r, axis=1) + a1
    ymn = jnp.min(y2r, axis=1) + a1
    ebv = eb_ref[...]
    es = ebv[0].reshape(1, C2)
    et = ebv[1].reshape(1, C2)
    h = jnp.maximum(_gelu(ymx * es + et), _gelu(ymn * es + et))
    enc = f + h
    h1 = _gelu(_bfdot(enc, w1_ref[...]) * sb1_ref[0:1, :] + sb1_ref[1:2, :])
    out_ref[...] = (_bfdot(h1, w2_ref[...]) * sb2_ref[0:1, :]
                    + sb2_ref[1:2, :])


def _edge_ffn_call(feat2, d, w1et, w2et, eb, w1t, sb1, w2t, sb2):
    return pl.pallas_call(
        _edge_ffn_body,
        grid=(EC,),
        in_specs=[
            pl.BlockSpec((NCH, C2), lambda i: (i, 0)),
            pl.BlockSpec((NCH * K, C2), lambda i: (i, 0)),
            pl.BlockSpec((C2, C2), lambda i: (0, 0)),
            pl.BlockSpec((C2, C2), lambda i: (0, 0)),
            pl.BlockSpec((8, C2), lambda i: (0, 0)),
            pl.BlockSpec((C2, C2), lambda i: (0, 0)),
            pl.BlockSpec((8, C2), lambda i: (0, 0)),
            pl.BlockSpec((C2, C2), lambda i: (0, 0)),
            pl.BlockSpec((8, C2), lambda i: (0, 0)),
        ],
        out_specs=pl.BlockSpec((NCH, C2), lambda i: (i, 0)),
        out_shape=jax.ShapeDtypeStruct((BN_, C2), jnp.float32),
        interpret=_INTERPRET,
    )(feat2, d, w1et, w2et, eb, w1t, sb1, w2t, sb2)


def _rows8(*vecs):
    out = jnp.zeros((8, vecs[0].shape[0]), jnp.float32)
    for i, v in enumerate(vecs):
        out = out.at[i, :].set(v)
    return out


def kernel(ts, params):
    p = params
    f32 = jnp.float32
    inv = 1.0 / jnp.sqrt(1.0 + EPS_BN)

    w1 = jnp.zeros((3, 128, C1), f32)
    for j in range(3):
        w1 = w1.at[j, :M, :].set(p["conv1_w"][:, :, j].T)
    w2 = jnp.zeros((3, 128, C2), f32)
    for j in range(3):
        w2 = w2.at[j, :C1, :].set(p["conv2_w"][:, :, j].T)
    b18 = jnp.broadcast_to(p["conv1_b"][None, :], (8, C1))
    b28 = jnp.broadcast_to(p["conv2_b"][None, :], (8, C2))

    xT = jnp.transpose(ts, (0, 2, 1))
    xpad = jnp.zeros((B, T + 8, 128), f32).at[:, 8:, :M].set(xT)

    feat = _conv_call(xpad, w1, b18, w2, b28)

    for i in range(2):
        s = p[f"g{i}_gamma"] * inv
        w1et = p[f"g{i}_We"][:, :C2].T
        w2et = p[f"g{i}_We"][:, C2:].T
        eb = _rows8(s, s * p[f"g{i}_be"] + p[f"g{i}_beta"])
        s1 = p[f"f{i}_g1"] * inv
        s2 = p[f"f{i}_g2"] * inv
        sb1 = _rows8(s1, s1 * p[f"f{i}_b1"] + p[f"f{i}_bt1"])
        sb2 = _rows8(s2, s2 * p[f"f{i}_b2"] + p[f"f{i}_bt2"])

        xn = _prep_call(feat)
        idx4 = _topk_call(xn)
        idx_flat = idx4[..., :K].reshape(NE)
        feat2 = feat.reshape(BN_, C2)
        d = _gather_diff_sc(feat2, idx_flat)
        feat = _edge_ffn_call(feat2, d, w1et, w2et, eb, p[f"f{i}_W1"].T,
                              sb1, p[f"f{i}_W2"].T, sb2).reshape(B, N, C2)

    return feat
